# baseline (speedup 1.0000x reference)
import jax
import jax.numpy as jnp
from jax import lax
from jax.experimental import pallas as pl
from jax.experimental.pallas import tpu as pltpu
from jax.experimental.pallas import tpu_sc as plsc

NC = 2
NS = 16
NW = NC * NS

TILE = 128


def _gating_body(x_ref, wg_ref, bg_ref, i0_ref, i1_ref, g0_ref, g1_ref,
                 r0_ref, r1_ref, cnt_ref, acc_ref):
    pid = pl.program_id(0)
    x = x_ref[...]
    logits = jnp.dot(x, wg_ref[...], preferred_element_type=jnp.float32)
    logits = logits + bg_ref[...]
    gb, e = logits.shape
    iota = lax.broadcasted_iota(jnp.int32, (gb, e), 1)
    m1 = jnp.max(logits, axis=1, keepdims=True)
    i1 = jnp.min(jnp.where(logits == m1, iota, e), axis=1, keepdims=True)
    l2 = jnp.where(iota == i1, -jnp.inf, logits)
    m2 = jnp.max(l2, axis=1, keepdims=True)
    i2 = jnp.min(jnp.where(l2 == m2, iota, e), axis=1, keepdims=True)
    e2 = jnp.exp(m2 - m1)
    s = 1.0 + e2
    i0_ref[...] = i1
    i1_ref[...] = i2
    g0_ref[...] = 1.0 / s
    g1_ref[...] = e2 / s

    @pl.when(pid == 0)
    def _():
        acc_ref[...] = jnp.zeros_like(acc_ref)

    oh0 = (i1 == iota).astype(jnp.bfloat16)
    oh1 = (i2 == iota).astype(jnp.bfloat16)
    row = lax.broadcasted_iota(jnp.int32, (gb, gb), 0)
    col = lax.broadcasted_iota(jnp.int32, (gb, gb), 1)
    tl = (row > col).astype(jnp.bfloat16)
    cum = jnp.dot(tl, oh0 + oh1, preferred_element_type=jnp.float32)
    base = acc_ref[...]
    oh0f = oh0.astype(jnp.float32)
    oh1f = oh1.astype(jnp.float32)
    r0 = jnp.sum((cum + base) * oh0f, axis=1, keepdims=True)
    r1 = jnp.sum((cum + base) * oh1f + oh0f * oh1f, axis=1, keepdims=True)
    r0_ref[...] = r0.astype(jnp.int32)
    r1_ref[...] = r1.astype(jnp.int32)
    total = base + jnp.sum(oh0f + oh1f, axis=0, keepdims=True)
    acc_ref[...] = total
    cnt_ref[...] = total.astype(jnp.int32)


def _gating(x, Wg, bg):
    n, d = x.shape
    e = Wg.shape[1]
    gb = 512
    return pl.pallas_call(
        _gating_body,
        grid=(n // gb,),
        in_specs=[
            pl.BlockSpec((gb, d), lambda i: (i, 0)),
            pl.BlockSpec((d, e), lambda i: (0, 0)),
            pl.BlockSpec((1, e), lambda i: (0, 0)),
        ],
        out_specs=[
            pl.BlockSpec((gb, 1), lambda i: (i, 0)),
            pl.BlockSpec((gb, 1), lambda i: (i, 0)),
            pl.BlockSpec((gb, 1), lambda i: (i, 0)),
            pl.BlockSpec((gb, 1), lambda i: (i, 0)),
            pl.BlockSpec((gb, 1), lambda i: (i, 0)),
            pl.BlockSpec((gb, 1), lambda i: (i, 0)),
            pl.BlockSpec((1, e), lambda i: (0, 0)),
        ],
        out_shape=[
            jax.ShapeDtypeStruct((n, 1), jnp.int32),
            jax.ShapeDtypeStruct((n, 1), jnp.int32),
            jax.ShapeDtypeStruct((n, 1), jnp.float32),
            jax.ShapeDtypeStruct((n, 1), jnp.float32),
            jax.ShapeDtypeStruct((n, 1), jnp.int32),
            jax.ShapeDtypeStruct((n, 1), jnp.int32),
            jax.ShapeDtypeStruct((1, e), jnp.int32),
        ],
        scratch_shapes=[pltpu.VMEM((1, e), jnp.float32)],
        compiler_params=pltpu.CompilerParams(
            dimension_semantics=("arbitrary",)),
    )(x, Wg, bg.reshape(1, e))


def _make_dispatch(n, dw, p):
    per_w = n // NW
    ch = 32
    n_ch = per_w // ch
    mesh = plsc.VectorSubcoreMesh(
        core_axis_name="c", subcore_axis_name="s",
        num_cores=NC, num_subcores=NS)

    def body(x_hbm, d0_hbm, d1_hbm, out_hbm, d0_v, d1_v, buf, sem):
        wid = lax.axis_index("s") * NC + lax.axis_index("c")
        pltpu.sync_copy(d0_hbm.at[wid], d0_v)
        pltpu.sync_copy(d1_hbm.at[wid], d1_v)
        pltpu.sync_copy(x_hbm.at[pl.ds(wid * per_w, per_w)], buf)
        copies = []
        for c in range(n_ch):
            src = buf.at[pl.ds(c * ch, ch)]
            copies.append(
                pltpu.async_copy(src, out_hbm.at[d0_v.at[c]], sem))
            copies.append(
                pltpu.async_copy(src, out_hbm.at[d1_v.at[c]], sem))
        for cp in copies:
            cp.wait()

    return pl.kernel(
        body,
        out_type=jax.ShapeDtypeStruct((p, dw), jnp.int32),
        mesh=mesh,
        scratch_types=[
            pltpu.VMEM((n_ch, ch), jnp.int32),
            pltpu.VMEM((n_ch, ch), jnp.int32),
            pltpu.VMEM((per_w, dw), jnp.int32),
            pltpu.SemaphoreType.DMA,
        ],
    )


def _mlp_body(te_ref, xd_ref, gs_ref, w1_ref, b1_ref, w2_ref, b2_ref,
              w3_ref, b3_ref, out_ref):
    xb = xd_ref[...]
    h = jnp.dot(xb, w1_ref[0], preferred_element_type=jnp.float32)
    h = jnp.maximum(h + b1_ref[0], 0.0).astype(jnp.bfloat16)
    h = jnp.dot(h, w2_ref[0], preferred_element_type=jnp.float32)
    h = jnp.maximum(h + b2_ref[0], 0.0).astype(jnp.bfloat16)
    o = jnp.dot(h, w3_ref[0], preferred_element_type=jnp.float32)
    out_ref[...] = (o + b3_ref[0]) * gs_ref[...]


def _mlp(te, xd, gs, W1, b1, W2, b2, W3, b3):
    p, d = xd.shape
    e, _, h = W1.shape
    o = W3.shape[2]
    b1 = b1.reshape(e, 1, h)
    b2 = b2.reshape(e, 1, h)
    b3 = b3.reshape(e, 1, o)
    nt = p // TILE
    grid_spec = pltpu.PrefetchScalarGridSpec(
        num_scalar_prefetch=1,
        grid=(nt,),
        in_specs=[
            pl.BlockSpec((TILE, d), lambda t, te: (t, 0)),
            pl.BlockSpec((TILE, 1), lambda t, te: (t, 0)),
            pl.BlockSpec((1, d, h), lambda t, te: (te[t], 0, 0)),
            pl.BlockSpec((1, 1, h), lambda t, te: (te[t], 0, 0)),
            pl.BlockSpec((1, h, h), lambda t, te: (te[t], 0, 0)),
            pl.BlockSpec((1, 1, h), lambda t, te: (te[t], 0, 0)),
            pl.BlockSpec((1, h, o), lambda t, te: (te[t], 0, 0)),
            pl.BlockSpec((1, 1, o), lambda t, te: (te[t], 0, 0)),
        ],
        out_specs=pl.BlockSpec((TILE, o), lambda t, te: (t, 0)),
    )
    return pl.pallas_call(
        _mlp_body,
        grid_spec=grid_spec,
        out_shape=jax.ShapeDtypeStruct((p, o), jnp.float32),
        compiler_params=pltpu.CompilerParams(
            dimension_semantics=("arbitrary",)),
    )(te, xd, gs, W1, b1, W2, b2, W3, b3)


def _make_combine(n, o, p):
    per_w = n // NW
    ch = 32
    n_ch = per_w // ch
    vec = 16
    mesh = plsc.VectorSubcoreMesh(
        core_axis_name="c", subcore_axis_name="s",
        num_cores=NC, num_subcores=NS)

    def body(y_hbm, d0_hbm, d1_hbm, out_hbm,
             d0_v, d1_v, buf0, buf1, sem0, sem1):
        wid = lax.axis_index("s") * NC + lax.axis_index("c")
        pltpu.sync_copy(d0_hbm.at[wid], d0_v)
        pltpu.sync_copy(d1_hbm.at[wid], d1_v)
        for c in range(n_ch):
            ca = pltpu.async_copy(y_hbm.at[d0_v.at[c]], buf0, sem0)
            cb = pltpu.async_copy(y_hbm.at[d1_v.at[c]], buf1, sem1)
            ca.wait()
            cb.wait()

            def row_add(r, _):
                for j in range(o // vec):
                    sl = pl.ds(j * vec, vec)
                    buf0[r, sl] = buf0[r, sl] + buf1[r, sl]
                return 0

            lax.fori_loop(0, ch, row_add, 0)
            pltpu.sync_copy(
                buf0, out_hbm.at[pl.ds(wid * per_w + c * ch, ch)])

    return pl.kernel(
        body,
        out_type=jax.ShapeDtypeStruct((n, o), jnp.float32),
        mesh=mesh,
        scratch_types=[
            pltpu.VMEM((n_ch, ch), jnp.int32),
            pltpu.VMEM((n_ch, ch), jnp.int32),
            pltpu.VMEM((ch, o), jnp.float32),
            pltpu.VMEM((ch, o), jnp.float32),
            pltpu.SemaphoreType.DMA,
            pltpu.SemaphoreType.DMA,
        ],
    )


def kernel(x, Wg, bg, W1, b1, W2, b2, W3, b3):
    n, d = x.shape
    e = Wg.shape[1]
    k = 2
    p = n * k + e * TILE
    o = W3.shape[2]

    i0, i1, g0, g1, r0, r1, counts = _gating(x, Wg, bg)

    counts = counts.reshape(e)
    tiles_per_e = (counts + TILE - 1) // TILE
    tile_bounds = jnp.cumsum(tiles_per_e)
    astart = (tile_bounds - tiles_per_e) * TILE
    dest0 = astart[i0[:, 0]] + r0[:, 0]
    dest1 = astart[i1[:, 0]] + r1[:, 0]
    gs = (jnp.zeros((p,), jnp.float32)
          .at[dest0].set(g0[:, 0]).at[dest1].set(g1[:, 0]).reshape(p, 1))
    te = jnp.minimum(
        jnp.searchsorted(tile_bounds, jnp.arange(p // TILE), side="right"),
        e - 1).astype(jnp.int32)

    xp = lax.bitcast_convert_type(
        x.astype(jnp.bfloat16).reshape(n, d // 2, 2), jnp.int32)
    per_w = n // NW
    d0r = dest0.reshape(NW, per_w // 32, 32)
    d1r = dest1.reshape(NW, per_w // 32, 32)
    xdp = _make_dispatch(n, d // 2, p)(xp, d0r, d1r)
    xd = lax.bitcast_convert_type(xdp, jnp.bfloat16).reshape(p, d)

    bf = jnp.bfloat16
    y = _mlp(te, xd, gs, W1.astype(bf), b1, W2.astype(bf), b2,
             W3.astype(bf), b3)

    return _make_combine(n, o, p)(y, d0r, d1r)


# speedup vs baseline: 1.1041x; 1.1041x over previous
import jax
import jax.numpy as jnp
from jax import lax
from jax.experimental import pallas as pl
from jax.experimental.pallas import tpu as pltpu
from jax.experimental.pallas import tpu_sc as plsc

NC = 2
NS = 16
NW = NC * NS

TILE = 128
VEC = 16


def _gating_body(x_ref, wg_ref, bg_ref, i0_ref, i1_ref, g0_ref, g1_ref,
                 r0_ref, r1_ref, cnt_ref, acc_ref):
    pid = pl.program_id(0)
    x = x_ref[...]
    logits = jnp.dot(x, wg_ref[...], preferred_element_type=jnp.float32)
    logits = logits + bg_ref[...]
    gb, e = logits.shape
    iota = lax.broadcasted_iota(jnp.int32, (gb, e), 1)
    m1 = jnp.max(logits, axis=1, keepdims=True)
    i1 = jnp.min(jnp.where(logits == m1, iota, e), axis=1, keepdims=True)
    l2 = jnp.where(iota == i1, -jnp.inf, logits)
    m2 = jnp.max(l2, axis=1, keepdims=True)
    i2 = jnp.min(jnp.where(l2 == m2, iota, e), axis=1, keepdims=True)
    e2 = jnp.exp(m2 - m1)
    s = 1.0 + e2
    i0_ref[...] = i1
    i1_ref[...] = i2
    g0_ref[...] = 1.0 / s
    g1_ref[...] = e2 / s

    @pl.when(pid == 0)
    def _():
        acc_ref[...] = jnp.zeros_like(acc_ref)

    oh0 = (i1 == iota).astype(jnp.bfloat16)
    oh1 = (i2 == iota).astype(jnp.bfloat16)
    row = lax.broadcasted_iota(jnp.int32, (gb, gb), 0)
    col = lax.broadcasted_iota(jnp.int32, (gb, gb), 1)
    tl = (row > col).astype(jnp.bfloat16)
    cum = jnp.dot(tl, oh0 + oh1, preferred_element_type=jnp.float32)
    base = acc_ref[...]
    oh0f = oh0.astype(jnp.float32)
    oh1f = oh1.astype(jnp.float32)
    r0 = jnp.sum((cum + base) * oh0f, axis=1, keepdims=True)
    r1 = jnp.sum((cum + base) * oh1f, axis=1, keepdims=True)
    r0_ref[...] = r0.astype(jnp.int32)
    r1_ref[...] = r1.astype(jnp.int32)
    total = base + jnp.sum(oh0f + oh1f, axis=0, keepdims=True)
    acc_ref[...] = total
    cnt_ref[...] = jnp.concatenate(
        [total, jnp.zeros((1, VEC - e), jnp.float32)], axis=1
    ).astype(jnp.int32)


def _gating(x, Wg, bg):
    n, d = x.shape
    e = Wg.shape[1]
    gb = 512
    return pl.pallas_call(
        _gating_body,
        grid=(n // gb,),
        in_specs=[
            pl.BlockSpec((gb, d), lambda i: (i, 0)),
            pl.BlockSpec((d, e), lambda i: (0, 0)),
            pl.BlockSpec((1, e), lambda i: (0, 0)),
        ],
        out_specs=[
            pl.BlockSpec((gb, 1), lambda i: (i, 0)),
            pl.BlockSpec((gb, 1), lambda i: (i, 0)),
            pl.BlockSpec((gb, 1), lambda i: (i, 0)),
            pl.BlockSpec((gb, 1), lambda i: (i, 0)),
            pl.BlockSpec((gb, 1), lambda i: (i, 0)),
            pl.BlockSpec((gb, 1), lambda i: (i, 0)),
            pl.BlockSpec((1, VEC), lambda i: (0, 0)),
        ],
        out_shape=[
            jax.ShapeDtypeStruct((n, 1), jnp.int32),
            jax.ShapeDtypeStruct((n, 1), jnp.int32),
            jax.ShapeDtypeStruct((n, 1), jnp.float32),
            jax.ShapeDtypeStruct((n, 1), jnp.float32),
            jax.ShapeDtypeStruct((n, 1), jnp.int32),
            jax.ShapeDtypeStruct((n, 1), jnp.int32),
            jax.ShapeDtypeStruct((1, VEC), jnp.int32),
        ],
        scratch_shapes=[pltpu.VMEM((1, e), jnp.float32)],
        compiler_params=pltpu.CompilerParams(
            dimension_semantics=("arbitrary",)),
    )(x, Wg, bg.reshape(1, e))


def _slot_dests(cnt_v, i0_v, i1_v, r0_v, r1_v, astart_v, d0i, d1i, per_w):
    cnt = cnt_v[...]
    ac = (cnt + (TILE - 1)) & jnp.int32(-TILE)
    astart_v[...] = plsc.cumsum(ac) - ac
    ch = d0i.shape[1]
    for v in range(per_w // VEC):
        sl = pl.ds(v * VEC, VEC)
        c, off = (v * VEC) // ch, (v * VEC) % ch
        osl = pl.ds(off, VEC)
        a0 = plsc.load_gather(astart_v, [i0_v[sl]])
        d0i[c, osl] = a0 + r0_v[sl]
        a1 = plsc.load_gather(astart_v, [i1_v[sl]])
        d1i[c, osl] = a1 + r1_v[sl]


def _make_dispatch(n, dw, p):
    per_w = n // NW
    ch = 32
    n_ch = per_w // ch
    mesh = plsc.VectorSubcoreMesh(
        core_axis_name="c", subcore_axis_name="s",
        num_cores=NC, num_subcores=NS)

    def body(x_hbm, i0_hbm, i1_hbm, r0_hbm, r1_hbm, cnt_hbm, out_hbm,
             i0_v, i1_v, r0_v, r1_v, cnt_v, astart_v, d0i, d1i, buf, sem):
        wid = lax.axis_index("s") * NC + lax.axis_index("c")
        pltpu.sync_copy(i0_hbm.at[wid], i0_v)
        pltpu.sync_copy(i1_hbm.at[wid], i1_v)
        pltpu.sync_copy(r0_hbm.at[wid], r0_v)
        pltpu.sync_copy(r1_hbm.at[wid], r1_v)
        pltpu.sync_copy(cnt_hbm, cnt_v)
        pltpu.sync_copy(x_hbm.at[pl.ds(wid * per_w, per_w)], buf)
        _slot_dests(cnt_v, i0_v, i1_v, r0_v, r1_v, astart_v, d0i, d1i,
                    per_w)
        copies = []
        for c in range(n_ch):
            src = buf.at[pl.ds(c * ch, ch)]
            copies.append(
                pltpu.async_copy(src, out_hbm.at[d0i.at[c]], sem))
            copies.append(
                pltpu.async_copy(src, out_hbm.at[d1i.at[c]], sem))
        for cp in copies:
            cp.wait()

    return pl.kernel(
        body,
        out_type=jax.ShapeDtypeStruct((p, dw), jnp.int32),
        mesh=mesh,
        compiler_params=pltpu.CompilerParams(needs_layout_passes=False),
        scratch_types=[
            pltpu.VMEM((per_w,), jnp.int32),
            pltpu.VMEM((per_w,), jnp.int32),
            pltpu.VMEM((per_w,), jnp.int32),
            pltpu.VMEM((per_w,), jnp.int32),
            pltpu.VMEM((VEC,), jnp.int32),
            pltpu.VMEM((VEC,), jnp.int32),
            pltpu.VMEM((n_ch, ch), jnp.int32),
            pltpu.VMEM((n_ch, ch), jnp.int32),
            pltpu.VMEM((per_w, dw), jnp.int32),
            pltpu.SemaphoreType.DMA,
        ],
    )


def _mlp_body(te_ref, xd_ref, w1_ref, b1_ref, w2_ref, b2_ref,
              w3_ref, b3_ref, out_ref):
    xb = xd_ref[...]
    h = jnp.dot(xb, w1_ref[0], preferred_element_type=jnp.float32)
    h = jnp.maximum(h + b1_ref[0], 0.0).astype(jnp.bfloat16)
    h = jnp.dot(h, w2_ref[0], preferred_element_type=jnp.float32)
    h = jnp.maximum(h + b2_ref[0], 0.0).astype(jnp.bfloat16)
    o = jnp.dot(h, w3_ref[0], preferred_element_type=jnp.float32)
    out_ref[...] = o + b3_ref[0]


def _mlp(te, xd, W1, b1, W2, b2, W3, b3):
    p, d = xd.shape
    e, _, h = W1.shape
    o = W3.shape[2]
    b1 = b1.reshape(e, 1, h)
    b2 = b2.reshape(e, 1, h)
    b3 = b3.reshape(e, 1, o)
    nt = p // TILE
    grid_spec = pltpu.PrefetchScalarGridSpec(
        num_scalar_prefetch=1,
        grid=(nt,),
        in_specs=[
            pl.BlockSpec((TILE, d), lambda t, te: (t, 0)),
            pl.BlockSpec((1, d, h), lambda t, te: (te[t], 0, 0)),
            pl.BlockSpec((1, 1, h), lambda t, te: (te[t], 0, 0)),
            pl.BlockSpec((1, h, h), lambda t, te: (te[t], 0, 0)),
            pl.BlockSpec((1, 1, h), lambda t, te: (te[t], 0, 0)),
            pl.BlockSpec((1, h, o), lambda t, te: (te[t], 0, 0)),
            pl.BlockSpec((1, 1, o), lambda t, te: (te[t], 0, 0)),
        ],
        out_specs=pl.BlockSpec((TILE, o), lambda t, te: (t, 0)),
    )
    return pl.pallas_call(
        _mlp_body,
        grid_spec=grid_spec,
        out_shape=jax.ShapeDtypeStruct((p, o), jnp.float32),
        compiler_params=pltpu.CompilerParams(
            dimension_semantics=("arbitrary",)),
    )(te, xd, W1, b1, W2, b2, W3, b3)


def _make_combine(n, o, p):
    per_w = n // NW
    ch = 32
    n_ch = per_w // ch
    mesh = plsc.VectorSubcoreMesh(
        core_axis_name="c", subcore_axis_name="s",
        num_cores=NC, num_subcores=NS)

    def body(y_hbm, i0_hbm, i1_hbm, r0_hbm, r1_hbm, g0_hbm, g1_hbm,
             cnt_hbm, out_hbm, i0_v, i1_v, r0_v, r1_v, g0_v, g1_v, cnt_v,
             astart_v, d0i, d1i, buf0, buf1, sem0, sem1):
        wid = lax.axis_index("s") * NC + lax.axis_index("c")
        pltpu.sync_copy(i0_hbm.at[wid], i0_v)
        pltpu.sync_copy(i1_hbm.at[wid], i1_v)
        pltpu.sync_copy(r0_hbm.at[wid], r0_v)
        pltpu.sync_copy(r1_hbm.at[wid], r1_v)
        pltpu.sync_copy(g0_hbm.at[wid], g0_v)
        pltpu.sync_copy(g1_hbm.at[wid], g1_v)
        pltpu.sync_copy(cnt_hbm, cnt_v)
        _slot_dests(cnt_v, i0_v, i1_v, r0_v, r1_v, astart_v, d0i, d1i,
                    per_w)
        for c in range(n_ch):
            ca = pltpu.async_copy(y_hbm.at[d0i.at[c]], buf0, sem0)
            cb = pltpu.async_copy(y_hbm.at[d1i.at[c]], buf1, sem1)
            ca.wait()
            cb.wait()

            def row_fma(r, _):
                t = c * ch + r
                g0b = plsc.load_gather(g0_v, [jnp.full((VEC,), t,
                                                       jnp.int32)])
                g1b = plsc.load_gather(g1_v, [jnp.full((VEC,), t,
                                                       jnp.int32)])
                for j in range(o // VEC):
                    sl = pl.ds(j * VEC, VEC)
                    buf0[r, sl] = buf0[r, sl] * g0b + buf1[r, sl] * g1b
                return 0

            lax.fori_loop(0, ch, row_fma, 0)
            pltpu.sync_copy(
                buf0, out_hbm.at[pl.ds(wid * per_w + c * ch, ch)])

    return pl.kernel(
        body,
        out_type=jax.ShapeDtypeStruct((n, o), jnp.float32),
        mesh=mesh,
        compiler_params=pltpu.CompilerParams(needs_layout_passes=False),
        scratch_types=[
            pltpu.VMEM((per_w,), jnp.int32),
            pltpu.VMEM((per_w,), jnp.int32),
            pltpu.VMEM((per_w,), jnp.int32),
            pltpu.VMEM((per_w,), jnp.int32),
            pltpu.VMEM((per_w,), jnp.float32),
            pltpu.VMEM((per_w,), jnp.float32),
            pltpu.VMEM((VEC,), jnp.int32),
            pltpu.VMEM((VEC,), jnp.int32),
            pltpu.VMEM((n_ch, ch), jnp.int32),
            pltpu.VMEM((n_ch, ch), jnp.int32),
            pltpu.VMEM((ch, o), jnp.float32),
            pltpu.VMEM((ch, o), jnp.float32),
            pltpu.SemaphoreType.DMA,
            pltpu.SemaphoreType.DMA,
        ],
    )


def kernel(x, Wg, bg, W1, b1, W2, b2, W3, b3):
    n, d = x.shape
    e = Wg.shape[1]
    k = 2
    p = n * k + e * TILE
    o = W3.shape[2]

    i0, i1, g0, g1, r0, r1, cnt16 = _gating(x, Wg, bg)
    cnt16 = cnt16.reshape(VEC)

    counts = cnt16[:e]
    tiles_per_e = (counts + TILE - 1) // TILE
    tile_bounds = jnp.cumsum(tiles_per_e)
    nt = p // TILE
    te = jnp.minimum(
        jnp.sum((jnp.arange(nt)[:, None] >= tile_bounds[None, :])
                .astype(jnp.int32), axis=1),
        e - 1).astype(jnp.int32)

    xp = lax.bitcast_convert_type(
        x.astype(jnp.bfloat16).reshape(n, d // 2, 2), jnp.int32)
    per_w = n // NW
    i0r = i0.reshape(NW, per_w)
    i1r = i1.reshape(NW, per_w)
    r0r = r0.reshape(NW, per_w)
    r1r = r1.reshape(NW, per_w)
    xdp = _make_dispatch(n, d // 2, p)(xp, i0r, i1r, r0r, r1r, cnt16)
    xd = lax.bitcast_convert_type(xdp, jnp.bfloat16).reshape(p, d)

    bf = jnp.bfloat16
    y = _mlp(te, xd, W1.astype(bf), b1, W2.astype(bf), b2,
             W3.astype(bf), b3)

    g0r = g0.reshape(NW, per_w)
    g1r = g1.reshape(NW, per_w)
    return _make_combine(n, o, p)(y, i0r, i1r, r0r, r1r, g0r, g1r, cnt16)


# speedup vs baseline: 2.1558x; 1.9525x over previous
import jax
import jax.numpy as jnp
from jax import lax
from jax.experimental import pallas as pl
from jax.experimental.pallas import tpu as pltpu
from jax.experimental.pallas import tpu_sc as plsc

NC = 2
NS = 16
NW = NC * NS

TILE = 128
VEC = 16


def _gating_body(x_ref, wg_ref, bg_ref, i0_ref, r0_ref, g0_ref,
                 cnt_ref, acc_ref):
    pid = pl.program_id(0)
    x = x_ref[...]
    logits = jnp.dot(x, wg_ref[...], preferred_element_type=jnp.float32)
    logits = logits + bg_ref[...]
    gb, e = logits.shape
    iota = lax.broadcasted_iota(jnp.int32, (gb, e), 1)
    m1 = jnp.max(logits, axis=1, keepdims=True)
    i1 = jnp.min(jnp.where(logits == m1, iota, e), axis=1, keepdims=True)
    l2 = jnp.where(iota == i1, -jnp.inf, logits)
    m2 = jnp.max(l2, axis=1, keepdims=True)
    i2 = jnp.min(jnp.where(l2 == m2, iota, e), axis=1, keepdims=True)
    e2 = jnp.exp(m2 - m1)
    s = 1.0 + e2
    g1v = 1.0 / s
    g2v = e2 / s

    @pl.when(pid == 0)
    def _():
        acc_ref[...] = jnp.zeros_like(acc_ref)

    oh0 = (i1 == iota).astype(jnp.bfloat16)
    oh1 = (i2 == iota).astype(jnp.bfloat16)
    row = lax.broadcasted_iota(jnp.int32, (gb, gb), 0)
    col = lax.broadcasted_iota(jnp.int32, (gb, gb), 1)
    tl = (row > col).astype(jnp.bfloat16)
    cum = jnp.dot(tl, oh0 + oh1, preferred_element_type=jnp.float32)
    base = acc_ref[...]
    oh0f = oh0.astype(jnp.float32)
    oh1f = oh1.astype(jnp.float32)
    r0 = jnp.sum((cum + base) * oh0f, axis=1, keepdims=True)
    r1 = jnp.sum((cum + base) * oh1f, axis=1, keepdims=True)

    def packw(a, b):
        return jnp.concatenate(
            [a.reshape(gb // 64, 64), b.reshape(gb // 64, 64)], axis=1)

    i0_ref[...] = packw(i1, i2)
    r0_ref[...] = packw(r0.astype(jnp.int32), r1.astype(jnp.int32))
    g0_ref[...] = packw(g1v, g2v)
    total = base + jnp.sum(oh0f + oh1f, axis=0, keepdims=True)
    acc_ref[...] = total
    cnt_ref[...] = jnp.concatenate(
        [total, jnp.zeros((1, VEC - e), jnp.float32)], axis=1
    ).astype(jnp.int32)


def _gating(x, Wg, bg):
    n, d = x.shape
    e = Wg.shape[1]
    gb = 512
    return pl.pallas_call(
        _gating_body,
        grid=(n // gb,),
        in_specs=[
            pl.BlockSpec((gb, d), lambda i: (i, 0)),
            pl.BlockSpec((d, e), lambda i: (0, 0)),
            pl.BlockSpec((1, e), lambda i: (0, 0)),
        ],
        out_specs=[
            pl.BlockSpec((gb // 64, 128), lambda i: (i, 0)),
            pl.BlockSpec((gb // 64, 128), lambda i: (i, 0)),
            pl.BlockSpec((gb // 64, 128), lambda i: (i, 0)),
            pl.BlockSpec((1, VEC), lambda i: (0, 0)),
        ],
        out_shape=[
            jax.ShapeDtypeStruct((n // 64, 128), jnp.int32),
            jax.ShapeDtypeStruct((n // 64, 128), jnp.int32),
            jax.ShapeDtypeStruct((n // 64, 128), jnp.float32),
            jax.ShapeDtypeStruct((1, VEC), jnp.int32),
        ],
        scratch_shapes=[pltpu.VMEM((1, e), jnp.float32)],
        compiler_params=pltpu.CompilerParams(
            dimension_semantics=("arbitrary",)),
    )(x, Wg, bg.reshape(1, e))


def _slot_dests(cnt_v, abi_v, abr_v, astart_v, d0i, d1i, per_w):
    cnt = cnt_v[...]
    ac = (cnt + (TILE - 1)) & jnp.int32(-TILE)
    astart_v[...] = plsc.cumsum(ac) - ac
    ch = d0i.shape[1]
    for v in range(per_w // VEC):
        sl0 = pl.ds(v * VEC, VEC)
        sl1 = pl.ds(per_w + v * VEC, VEC)
        c, off = (v * VEC) // ch, (v * VEC) % ch
        osl = pl.ds(off, VEC)
        a0 = plsc.load_gather(astart_v, [abi_v[sl0]])
        d0i[c, osl] = a0 + abr_v[sl0]
        a1 = plsc.load_gather(astart_v, [abi_v[sl1]])
        d1i[c, osl] = a1 + abr_v[sl1]


def _make_dispatch(n, d, p):
    per_w = n // NW
    ch = 32
    n_ch = per_w // ch
    mesh = plsc.VectorSubcoreMesh(
        core_axis_name="c", subcore_axis_name="s",
        num_cores=NC, num_subcores=NS)

    def body(x_hbm, abi_hbm, abr_hbm, cnt_hbm, out_hbm,
             abi_v, abr_v, cnt_v, astart_v, d0i, d1i, buf, sem):
        wid = lax.axis_index("s") * NC + lax.axis_index("c")
        pltpu.sync_copy(abi_hbm.at[wid], abi_v)
        pltpu.sync_copy(abr_hbm.at[wid], abr_v)
        pltpu.sync_copy(cnt_hbm, cnt_v)
        pltpu.sync_copy(x_hbm.at[pl.ds(wid * per_w, per_w)], buf)
        _slot_dests(cnt_v, abi_v, abr_v, astart_v, d0i, d1i, per_w)
        copies = []
        for c in range(n_ch):
            src = buf.at[pl.ds(c * ch, ch)]
            copies.append(
                pltpu.async_copy(src, out_hbm.at[d0i.at[c]], sem))
            copies.append(
                pltpu.async_copy(src, out_hbm.at[d1i.at[c]], sem))
        for cp in copies:
            cp.wait()

    return pl.kernel(
        body,
        out_type=jax.ShapeDtypeStruct((p, d), jnp.float32),
        mesh=mesh,
        compiler_params=pltpu.CompilerParams(needs_layout_passes=False),
        scratch_types=[
            pltpu.VMEM((2 * per_w,), jnp.int32),
            pltpu.VMEM((2 * per_w,), jnp.int32),
            pltpu.VMEM((VEC,), jnp.int32),
            pltpu.VMEM((VEC,), jnp.int32),
            pltpu.VMEM((n_ch, ch), jnp.int32),
            pltpu.VMEM((n_ch, ch), jnp.int32),
            pltpu.VMEM((per_w, d), jnp.float32),
            pltpu.SemaphoreType.DMA,
        ],
    )


def _mlp_body(te_ref, xd_ref, w1_ref, b1_ref, w2_ref, b2_ref,
              w3_ref, b3_ref, out_ref):
    xb = xd_ref[...].astype(jnp.bfloat16)
    h = jnp.dot(xb, w1_ref[0], preferred_element_type=jnp.float32)
    h = jnp.maximum(h + b1_ref[0], 0.0).astype(jnp.bfloat16)
    h = jnp.dot(h, w2_ref[0], preferred_element_type=jnp.float32)
    h = jnp.maximum(h + b2_ref[0], 0.0).astype(jnp.bfloat16)
    o = jnp.dot(h, w3_ref[0], preferred_element_type=jnp.float32)
    out_ref[...] = o + b3_ref[0]


def _mlp(te, xd, W1, b1, W2, b2, W3, b3):
    p, d = xd.shape
    e, _, h = W1.shape
    o = W3.shape[2]
    b1 = b1.reshape(e, 1, h)
    b2 = b2.reshape(e, 1, h)
    b3 = b3.reshape(e, 1, o)
    nt = p // TILE
    grid_spec = pltpu.PrefetchScalarGridSpec(
        num_scalar_prefetch=1,
        grid=(nt,),
        in_specs=[
            pl.BlockSpec((TILE, d), lambda t, te: (t, 0)),
            pl.BlockSpec((1, d, h), lambda t, te: (te[t], 0, 0)),
            pl.BlockSpec((1, 1, h), lambda t, te: (te[t], 0, 0)),
            pl.BlockSpec((1, h, h), lambda t, te: (te[t], 0, 0)),
            pl.BlockSpec((1, 1, h), lambda t, te: (te[t], 0, 0)),
            pl.BlockSpec((1, h, o), lambda t, te: (te[t], 0, 0)),
            pl.BlockSpec((1, 1, o), lambda t, te: (te[t], 0, 0)),
        ],
        out_specs=pl.BlockSpec((TILE, o), lambda t, te: (t, 0)),
    )
    return pl.pallas_call(
        _mlp_body,
        grid_spec=grid_spec,
        out_shape=jax.ShapeDtypeStruct((p, o), jnp.float32),
        compiler_params=pltpu.CompilerParams(
            dimension_semantics=("arbitrary",)),
    )(te, xd, W1, b1, W2, b2, W3, b3)


def _make_combine(n, o, p):
    per_w = n // NW
    ch = 32
    n_ch = per_w // ch
    mesh = plsc.VectorSubcoreMesh(
        core_axis_name="c", subcore_axis_name="s",
        num_cores=NC, num_subcores=NS)

    def body(y_hbm, abi_hbm, abr_hbm, abg_hbm, cnt_hbm, out_hbm,
             abi_v, abr_v, abg_v, cnt_v,
             astart_v, d0i, d1i, buf0, buf1, sem0, sem1):
        wid = lax.axis_index("s") * NC + lax.axis_index("c")
        pltpu.sync_copy(abi_hbm.at[wid], abi_v)
        pltpu.sync_copy(abr_hbm.at[wid], abr_v)
        pltpu.sync_copy(abg_hbm.at[wid], abg_v)
        pltpu.sync_copy(cnt_hbm, cnt_v)
        _slot_dests(cnt_v, abi_v, abr_v, astart_v, d0i, d1i, per_w)
        for c in range(n_ch):
            ca = pltpu.async_copy(y_hbm.at[d0i.at[c]], buf0, sem0)
            cb = pltpu.async_copy(y_hbm.at[d1i.at[c]], buf1, sem1)
            ca.wait()
            cb.wait()

            def row_fma(r, _):
                t = c * ch + r
                g0b = plsc.load_gather(abg_v, [jnp.full((VEC,), t,
                                                        jnp.int32)])
                g1b = plsc.load_gather(abg_v, [jnp.full((VEC,), per_w + t,
                                                        jnp.int32)])
                for j in range(o // VEC):
                    sl = pl.ds(j * VEC, VEC)
                    buf0[r, sl] = buf0[r, sl] * g0b + buf1[r, sl] * g1b
                return 0

            lax.fori_loop(0, ch, row_fma, 0)
            pltpu.sync_copy(
                buf0, out_hbm.at[pl.ds(wid * per_w + c * ch, ch)])

    return pl.kernel(
        body,
        out_type=jax.ShapeDtypeStruct((n, o), jnp.float32),
        mesh=mesh,
        compiler_params=pltpu.CompilerParams(needs_layout_passes=False),
        scratch_types=[
            pltpu.VMEM((2 * per_w,), jnp.int32),
            pltpu.VMEM((2 * per_w,), jnp.int32),
            pltpu.VMEM((2 * per_w,), jnp.float32),
            pltpu.VMEM((VEC,), jnp.int32),
            pltpu.VMEM((VEC,), jnp.int32),
            pltpu.VMEM((n_ch, ch), jnp.int32),
            pltpu.VMEM((n_ch, ch), jnp.int32),
            pltpu.VMEM((ch, o), jnp.float32),
            pltpu.VMEM((ch, o), jnp.float32),
            pltpu.SemaphoreType.DMA,
            pltpu.SemaphoreType.DMA,
        ],
    )


def kernel(x, Wg, bg, W1, b1, W2, b2, W3, b3):
    n, d = x.shape
    e = Wg.shape[1]
    k = 2
    p = n * k + e * TILE
    o = W3.shape[2]

    abi, abr, abg, cnt16 = _gating(x, Wg, bg)
    cnt16 = cnt16.reshape(VEC)

    counts = cnt16[:e]
    tiles_per_e = (counts + TILE - 1) // TILE
    tile_bounds = jnp.cumsum(tiles_per_e)
    nt = p // TILE
    te = jnp.minimum(
        jnp.sum((jnp.arange(nt)[:, None] >= tile_bounds[None, :])
                .astype(jnp.int32), axis=1),
        e - 1).astype(jnp.int32)

    xd = _make_dispatch(n, d, p)(x, abi, abr, cnt16)

    bf = jnp.bfloat16
    y = _mlp(te, xd, W1.astype(bf), b1, W2.astype(bf), b2,
             W3.astype(bf), b3)

    return _make_combine(n, o, p)(y, abi, abr, abg, cnt16)


# speedup vs baseline: 2.5353x; 1.1760x over previous
import jax
import jax.numpy as jnp
from jax import lax
from jax.experimental import pallas as pl
from jax.experimental.pallas import tpu as pltpu
from jax.experimental.pallas import tpu_sc as plsc

NC = 2
NS = 16
NW = NC * NS

TILE = 128
VEC = 16


def _gating_body(x_ref, wg_ref, bg_ref, i0_ref, r0_ref, g0_ref,
                 cnt_ref, acc_ref):
    pid = pl.program_id(0)
    x = x_ref[...]
    logits = jnp.dot(x, wg_ref[...], preferred_element_type=jnp.float32)
    logits = logits + bg_ref[...]
    gb, e = logits.shape
    iota = lax.broadcasted_iota(jnp.int32, (gb, e), 1)
    m1 = jnp.max(logits, axis=1, keepdims=True)
    i1 = jnp.min(jnp.where(logits == m1, iota, e), axis=1, keepdims=True)
    l2 = jnp.where(iota == i1, -jnp.inf, logits)
    m2 = jnp.max(l2, axis=1, keepdims=True)
    i2 = jnp.min(jnp.where(l2 == m2, iota, e), axis=1, keepdims=True)
    e2 = jnp.exp(m2 - m1)
    s = 1.0 + e2
    g1v = 1.0 / s
    g2v = e2 / s

    @pl.when(pid == 0)
    def _():
        acc_ref[...] = jnp.zeros_like(acc_ref)

    oh0 = (i1 == iota).astype(jnp.bfloat16)
    oh1 = (i2 == iota).astype(jnp.bfloat16)
    row = lax.broadcasted_iota(jnp.int32, (gb, gb), 0)
    col = lax.broadcasted_iota(jnp.int32, (gb, gb), 1)
    tl = (row > col).astype(jnp.bfloat16)
    cum = jnp.dot(tl, oh0 + oh1, preferred_element_type=jnp.float32)
    base = acc_ref[...]
    oh0f = oh0.astype(jnp.float32)
    oh1f = oh1.astype(jnp.float32)
    r0 = jnp.sum((cum + base) * oh0f, axis=1, keepdims=True)
    r1 = jnp.sum((cum + base) * oh1f, axis=1, keepdims=True)

    def packw(a, b):
        return jnp.concatenate(
            [a.reshape(gb // 64, 64), b.reshape(gb // 64, 64)], axis=1)

    i0_ref[...] = packw(i1, i2)
    r0_ref[...] = packw(r0.astype(jnp.int32), r1.astype(jnp.int32))
    g0_ref[...] = packw(g1v, g2v)
    total = base + jnp.sum(oh0f + oh1f, axis=0, keepdims=True)
    acc_ref[...] = total
    cnt_ref[...] = jnp.concatenate(
        [total, jnp.zeros((1, VEC - e), jnp.float32)], axis=1
    ).astype(jnp.int32)


def _gating(x, Wg, bg):
    n, d = x.shape
    e = Wg.shape[1]
    gb = 512
    return pl.pallas_call(
        _gating_body,
        grid=(n // gb,),
        in_specs=[
            pl.BlockSpec((gb, d), lambda i: (i, 0)),
            pl.BlockSpec((d, e), lambda i: (0, 0)),
            pl.BlockSpec((1, e), lambda i: (0, 0)),
        ],
        out_specs=[
            pl.BlockSpec((gb // 64, 128), lambda i: (i, 0)),
            pl.BlockSpec((gb // 64, 128), lambda i: (i, 0)),
            pl.BlockSpec((gb // 64, 128), lambda i: (i, 0)),
            pl.BlockSpec((1, VEC), lambda i: (0, 0)),
        ],
        out_shape=[
            jax.ShapeDtypeStruct((n // 64, 128), jnp.int32),
            jax.ShapeDtypeStruct((n // 64, 128), jnp.int32),
            jax.ShapeDtypeStruct((n // 64, 128), jnp.float32),
            jax.ShapeDtypeStruct((1, VEC), jnp.int32),
        ],
        scratch_shapes=[pltpu.VMEM((1, e), jnp.float32)],
        compiler_params=pltpu.CompilerParams(
            dimension_semantics=("arbitrary",)),
    )(x, Wg, bg.reshape(1, e))


def _slot_dests(cnt_v, abi_v, abr_v, astart_v, d0i, d1i, per_w):
    cnt = cnt_v[...]
    ac = (cnt + (TILE - 1)) & jnp.int32(-TILE)
    astart_v[...] = plsc.cumsum(ac) - ac
    ch = d0i.shape[1]
    for v in range(per_w // VEC):
        sl0 = pl.ds(v * VEC, VEC)
        sl1 = pl.ds(per_w + v * VEC, VEC)
        c, off = (v * VEC) // ch, (v * VEC) % ch
        osl = pl.ds(off, VEC)
        a0 = plsc.load_gather(astart_v, [abi_v[sl0]])
        d0i[c, osl] = a0 + abr_v[sl0]
        a1 = plsc.load_gather(astart_v, [abi_v[sl1]])
        d1i[c, osl] = a1 + abr_v[sl1]


def _make_dispatch(n, d, p):
    per_w = n // NW
    ch = 32
    n_ch = per_w // ch
    mesh = plsc.VectorSubcoreMesh(
        core_axis_name="c", subcore_axis_name="s",
        num_cores=NC, num_subcores=NS)

    def body(x_hbm, abi_hbm, abr_hbm, cnt_hbm, out_hbm,
             abi_v, abr_v, cnt_v, astart_v, d0i, d1i, buf, sem):
        wid = lax.axis_index("s") * NC + lax.axis_index("c")
        pltpu.sync_copy(abi_hbm.at[wid], abi_v)
        pltpu.sync_copy(abr_hbm.at[wid], abr_v)
        pltpu.sync_copy(cnt_hbm, cnt_v)
        pltpu.sync_copy(x_hbm.at[pl.ds(wid * per_w, per_w)], buf)
        _slot_dests(cnt_v, abi_v, abr_v, astart_v, d0i, d1i, per_w)
        copies = []
        for c in range(n_ch):
            src = buf.at[pl.ds(c * ch, ch)]
            copies.append(
                pltpu.async_copy(src, out_hbm.at[d0i.at[c]], sem))
            copies.append(
                pltpu.async_copy(src, out_hbm.at[d1i.at[c]], sem))
        for cp in copies:
            cp.wait()

    return pl.kernel(
        body,
        out_type=jax.ShapeDtypeStruct((p, d), jnp.float32),
        mesh=mesh,
        compiler_params=pltpu.CompilerParams(needs_layout_passes=False),
        scratch_types=[
            pltpu.VMEM((2 * per_w,), jnp.int32),
            pltpu.VMEM((2 * per_w,), jnp.int32),
            pltpu.VMEM((VEC,), jnp.int32),
            pltpu.VMEM((VEC,), jnp.int32),
            pltpu.VMEM((n_ch, ch), jnp.int32),
            pltpu.VMEM((n_ch, ch), jnp.int32),
            pltpu.VMEM((per_w, d), jnp.float32),
            pltpu.SemaphoreType.DMA,
        ],
    )


def _mlp_body(te_ref, xd_ref, w1_ref, b1_ref, w2_ref, b2_ref,
              w3_ref, b3_ref, out_ref, w1b, w2b, w3b):
    t = pl.program_id(0)
    is_new = jnp.logical_or(
        t == 0, te_ref[t] != te_ref[jnp.maximum(t - 1, 0)])

    @pl.when(is_new)
    def _():
        w1b[...] = w1_ref[0].astype(jnp.bfloat16)
        w2b[...] = w2_ref[0].astype(jnp.bfloat16)
        w3b[...] = w3_ref[0].astype(jnp.bfloat16)

    xb = xd_ref[...].astype(jnp.bfloat16)
    h = jnp.dot(xb, w1b[...], preferred_element_type=jnp.float32)
    h = jnp.maximum(h + b1_ref[0], 0.0).astype(jnp.bfloat16)
    h = jnp.dot(h, w2b[...], preferred_element_type=jnp.float32)
    h = jnp.maximum(h + b2_ref[0], 0.0).astype(jnp.bfloat16)
    o = jnp.dot(h, w3b[...], preferred_element_type=jnp.float32)
    out_ref[...] = o + b3_ref[0]


def _mlp(te, xd, W1, b1, W2, b2, W3, b3):
    p, d = xd.shape
    e, _, h = W1.shape
    o = W3.shape[2]
    b1 = b1.reshape(e, 1, h)
    b2 = b2.reshape(e, 1, h)
    b3 = b3.reshape(e, 1, o)
    nt = p // TILE
    grid_spec = pltpu.PrefetchScalarGridSpec(
        num_scalar_prefetch=1,
        grid=(nt,),
        in_specs=[
            pl.BlockSpec((TILE, d), lambda t, te: (t, 0)),
            pl.BlockSpec((1, d, h), lambda t, te: (te[t], 0, 0)),
            pl.BlockSpec((1, 1, h), lambda t, te: (te[t], 0, 0)),
            pl.BlockSpec((1, h, h), lambda t, te: (te[t], 0, 0)),
            pl.BlockSpec((1, 1, h), lambda t, te: (te[t], 0, 0)),
            pl.BlockSpec((1, h, o), lambda t, te: (te[t], 0, 0)),
            pl.BlockSpec((1, 1, o), lambda t, te: (te[t], 0, 0)),
        ],
        out_specs=pl.BlockSpec((TILE, o), lambda t, te: (t, 0)),
        scratch_shapes=[
            pltpu.VMEM((d, h), jnp.bfloat16),
            pltpu.VMEM((h, h), jnp.bfloat16),
            pltpu.VMEM((h, o), jnp.bfloat16),
        ],
    )
    return pl.pallas_call(
        _mlp_body,
        grid_spec=grid_spec,
        out_shape=jax.ShapeDtypeStruct((p, o), jnp.float32),
        compiler_params=pltpu.CompilerParams(
            dimension_semantics=("arbitrary",)),
    )(te, xd, W1, b1, W2, b2, W3, b3)


def _make_combine(n, o, p):
    per_w = n // NW
    ch = 32
    n_ch = per_w // ch
    mesh = plsc.VectorSubcoreMesh(
        core_axis_name="c", subcore_axis_name="s",
        num_cores=NC, num_subcores=NS)

    def body(y_hbm, abi_hbm, abr_hbm, abg_hbm, cnt_hbm, out_hbm,
             abi_v, abr_v, abg_v, cnt_v,
             astart_v, d0i, d1i, buf0, buf1, sem0, sem1):
        wid = lax.axis_index("s") * NC + lax.axis_index("c")
        pltpu.sync_copy(abi_hbm.at[wid], abi_v)
        pltpu.sync_copy(abr_hbm.at[wid], abr_v)
        pltpu.sync_copy(abg_hbm.at[wid], abg_v)
        pltpu.sync_copy(cnt_hbm, cnt_v)
        _slot_dests(cnt_v, abi_v, abr_v, astart_v, d0i, d1i, per_w)
        for c in range(n_ch):
            ca = pltpu.async_copy(y_hbm.at[d0i.at[c]], buf0, sem0)
            cb = pltpu.async_copy(y_hbm.at[d1i.at[c]], buf1, sem1)
            ca.wait()
            cb.wait()

            def row_fma(r, _):
                t = c * ch + r
                g0b = plsc.load_gather(abg_v, [jnp.full((VEC,), t,
                                                        jnp.int32)])
                g1b = plsc.load_gather(abg_v, [jnp.full((VEC,), per_w + t,
                                                        jnp.int32)])
                for j in range(o // VEC):
                    sl = pl.ds(j * VEC, VEC)
                    buf0[r, sl] = buf0[r, sl] * g0b + buf1[r, sl] * g1b
                return 0

            lax.fori_loop(0, ch, row_fma, 0)
            pltpu.sync_copy(
                buf0, out_hbm.at[pl.ds(wid * per_w + c * ch, ch)])

    return pl.kernel(
        body,
        out_type=jax.ShapeDtypeStruct((n, o), jnp.float32),
        mesh=mesh,
        compiler_params=pltpu.CompilerParams(needs_layout_passes=False),
        scratch_types=[
            pltpu.VMEM((2 * per_w,), jnp.int32),
            pltpu.VMEM((2 * per_w,), jnp.int32),
            pltpu.VMEM((2 * per_w,), jnp.float32),
            pltpu.VMEM((VEC,), jnp.int32),
            pltpu.VMEM((VEC,), jnp.int32),
            pltpu.VMEM((n_ch, ch), jnp.int32),
            pltpu.VMEM((n_ch, ch), jnp.int32),
            pltpu.VMEM((ch, o), jnp.float32),
            pltpu.VMEM((ch, o), jnp.float32),
            pltpu.SemaphoreType.DMA,
            pltpu.SemaphoreType.DMA,
        ],
    )


def kernel(x, Wg, bg, W1, b1, W2, b2, W3, b3):
    n, d = x.shape
    e = Wg.shape[1]
    k = 2
    p = n * k + e * TILE
    o = W3.shape[2]

    abi, abr, abg, cnt16 = _gating(x, Wg, bg)
    cnt16 = cnt16.reshape(VEC)

    counts = cnt16[:e]
    tiles_per_e = (counts + TILE - 1) // TILE
    tile_bounds = jnp.cumsum(tiles_per_e)
    nt = p // TILE
    te = jnp.minimum(
        jnp.sum((jnp.arange(nt)[:, None] >= tile_bounds[None, :])
                .astype(jnp.int32), axis=1),
        e - 1).astype(jnp.int32)

    xd = _make_dispatch(n, d, p)(x, abi, abr, cnt16)

    y = _mlp(te, xd, W1, b1, W2, b2, W3, b3)

    return _make_combine(n, o, p)(y, abi, abr, abg, cnt16)


# speedup vs baseline: 2.5533x; 1.0071x over previous
import jax
import jax.numpy as jnp
from jax import lax
from jax.experimental import pallas as pl
from jax.experimental.pallas import tpu as pltpu
from jax.experimental.pallas import tpu_sc as plsc

NC = 2
NS = 16
NW = NC * NS

TILE = 128
VEC = 16


def _gating_body(x_ref, wg_ref, bg_ref, i0_ref, r0_ref, g0_ref,
                 cnt_ref, te_ref, acc_ref):
    pid = pl.program_id(0)
    x = x_ref[...]
    logits = jnp.dot(x, wg_ref[...], preferred_element_type=jnp.float32)
    logits = logits + bg_ref[...]
    gb, e = logits.shape
    iota = lax.broadcasted_iota(jnp.int32, (gb, e), 1)
    m1 = jnp.max(logits, axis=1, keepdims=True)
    i1 = jnp.min(jnp.where(logits == m1, iota, e), axis=1, keepdims=True)
    l2 = jnp.where(iota == i1, -jnp.inf, logits)
    m2 = jnp.max(l2, axis=1, keepdims=True)
    i2 = jnp.min(jnp.where(l2 == m2, iota, e), axis=1, keepdims=True)
    e2 = jnp.exp(m2 - m1)
    s = 1.0 + e2
    g1v = 1.0 / s
    g2v = e2 / s

    @pl.when(pid == 0)
    def _():
        acc_ref[...] = jnp.zeros_like(acc_ref)

    oh0 = (i1 == iota).astype(jnp.bfloat16)
    oh1 = (i2 == iota).astype(jnp.bfloat16)
    row = lax.broadcasted_iota(jnp.int32, (gb, gb), 0)
    col = lax.broadcasted_iota(jnp.int32, (gb, gb), 1)
    tl = (row > col).astype(jnp.bfloat16)
    cum = jnp.dot(tl, oh0 + oh1, preferred_element_type=jnp.float32)
    base = acc_ref[...]
    oh0f = oh0.astype(jnp.float32)
    oh1f = oh1.astype(jnp.float32)
    r0 = jnp.sum((cum + base) * oh0f, axis=1, keepdims=True)
    r1 = jnp.sum((cum + base) * oh1f, axis=1, keepdims=True)

    def packw(a, b):
        return jnp.concatenate(
            [a.reshape(gb // 64, 64), b.reshape(gb // 64, 64)], axis=1)

    i0_ref[...] = packw(i1, i2)
    r0_ref[...] = packw(r0.astype(jnp.int32), r1.astype(jnp.int32))
    g0_ref[...] = packw(g1v, g2v)
    total = base + jnp.sum(oh0f + oh1f, axis=0, keepdims=True)
    acc_ref[...] = total
    cnt_ref[...] = jnp.concatenate(
        [total, jnp.zeros((1, VEC - e), jnp.float32)], axis=1
    ).astype(jnp.int32)

    ac_t = jnp.floor((total + (TILE - 1)) * (1.0 / TILE))
    tri = (lax.broadcasted_iota(jnp.int32, (e, e), 0)
           <= lax.broadcasted_iota(jnp.int32, (e, e), 1)).astype(
               jnp.float32)
    bounds = jnp.dot(ac_t, tri, preferred_element_type=jnp.float32)
    tt = lax.broadcasted_iota(jnp.int32, (1, 64), 1).astype(jnp.float32)
    tev = sum((tt >= bounds[:, j:j + 1]).astype(jnp.int32)
              for j in range(e))
    te_ref[...] = jnp.minimum(tev, e - 1)


def _gating(x, Wg, bg):
    n, d = x.shape
    e = Wg.shape[1]
    gb = 512
    return pl.pallas_call(
        _gating_body,
        grid=(n // gb,),
        in_specs=[
            pl.BlockSpec((gb, d), lambda i: (i, 0)),
            pl.BlockSpec((d, e), lambda i: (0, 0)),
            pl.BlockSpec((1, e), lambda i: (0, 0)),
        ],
        out_specs=[
            pl.BlockSpec((gb // 64, 128), lambda i: (i, 0)),
            pl.BlockSpec((gb // 64, 128), lambda i: (i, 0)),
            pl.BlockSpec((gb // 64, 128), lambda i: (i, 0)),
            pl.BlockSpec((1, VEC), lambda i: (0, 0)),
            pl.BlockSpec((1, 64), lambda i: (0, 0)),
        ],
        out_shape=[
            jax.ShapeDtypeStruct((n // 64, 128), jnp.int32),
            jax.ShapeDtypeStruct((n // 64, 128), jnp.int32),
            jax.ShapeDtypeStruct((n // 64, 128), jnp.float32),
            jax.ShapeDtypeStruct((1, VEC), jnp.int32),
            jax.ShapeDtypeStruct((1, 64), jnp.int32),
        ],
        scratch_shapes=[pltpu.VMEM((1, e), jnp.float32)],
        compiler_params=pltpu.CompilerParams(
            dimension_semantics=("arbitrary",)),
    )(x, Wg, bg.reshape(1, e))


def _slot_dests(cnt_v, abi_v, abr_v, astart_v, d0i, d1i, per_w):
    cnt = cnt_v[...]
    ac = (cnt + (TILE - 1)) & jnp.int32(-TILE)
    astart_v[...] = plsc.cumsum(ac) - ac
    ch = d0i.shape[1]
    for v in range(per_w // VEC):
        sl0 = pl.ds(v * VEC, VEC)
        sl1 = pl.ds(per_w + v * VEC, VEC)
        c, off = (v * VEC) // ch, (v * VEC) % ch
        osl = pl.ds(off, VEC)
        a0 = plsc.load_gather(astart_v, [abi_v[sl0]])
        d0i[c, osl] = a0 + abr_v[sl0]
        a1 = plsc.load_gather(astart_v, [abi_v[sl1]])
        d1i[c, osl] = a1 + abr_v[sl1]


def _make_dispatch(n, d, p):
    per_w = n // NW
    ch = 32
    n_ch = per_w // ch
    mesh = plsc.VectorSubcoreMesh(
        core_axis_name="c", subcore_axis_name="s",
        num_cores=NC, num_subcores=NS)

    def body(x_hbm, abi_hbm, abr_hbm, cnt_hbm, out_hbm,
             abi_v, abr_v, cnt_v, astart_v, d0i, d1i, buf, sem, semx):
        wid = lax.axis_index("s") * NC + lax.axis_index("c")
        cx = pltpu.async_copy(x_hbm.at[pl.ds(wid * per_w, per_w)], buf,
                              semx)
        pltpu.sync_copy(abi_hbm.at[wid], abi_v)
        pltpu.sync_copy(abr_hbm.at[wid], abr_v)
        pltpu.sync_copy(cnt_hbm, cnt_v)
        _slot_dests(cnt_v, abi_v, abr_v, astart_v, d0i, d1i, per_w)
        cx.wait()
        copies = []
        for c in range(n_ch):
            src = buf.at[pl.ds(c * ch, ch)]
            copies.append(
                pltpu.async_copy(src, out_hbm.at[d0i.at[c]], sem))
            copies.append(
                pltpu.async_copy(src, out_hbm.at[d1i.at[c]], sem))
        for cp in copies:
            cp.wait()

    return pl.kernel(
        body,
        out_type=jax.ShapeDtypeStruct((p, d), jnp.float32),
        mesh=mesh,
        compiler_params=pltpu.CompilerParams(needs_layout_passes=False),
        scratch_types=[
            pltpu.VMEM((2 * per_w,), jnp.int32),
            pltpu.VMEM((2 * per_w,), jnp.int32),
            pltpu.VMEM((VEC,), jnp.int32),
            pltpu.VMEM((VEC,), jnp.int32),
            pltpu.VMEM((n_ch, ch), jnp.int32),
            pltpu.VMEM((n_ch, ch), jnp.int32),
            pltpu.VMEM((per_w, d), jnp.float32),
            pltpu.SemaphoreType.DMA,
            pltpu.SemaphoreType.DMA,
        ],
    )


def _mlp_body(te_ref, xd_ref, w1_ref, b1_ref, w2_ref, b2_ref,
              w3_ref, b3_ref, out_ref, w1b, w2b, w3b):
    t = pl.program_id(0)
    is_new = jnp.logical_or(
        t == 0, te_ref[t] != te_ref[jnp.maximum(t - 1, 0)])

    @pl.when(is_new)
    def _():
        w1b[...] = w1_ref[0].astype(jnp.bfloat16)
        w2b[...] = w2_ref[0].astype(jnp.bfloat16)
        w3b[...] = w3_ref[0].astype(jnp.bfloat16)

    xb = xd_ref[...].astype(jnp.bfloat16)
    h = jnp.dot(xb, w1b[...], preferred_element_type=jnp.float32)
    h = jnp.maximum(h + b1_ref[0], 0.0).astype(jnp.bfloat16)
    h = jnp.dot(h, w2b[...], preferred_element_type=jnp.float32)
    h = jnp.maximum(h + b2_ref[0], 0.0).astype(jnp.bfloat16)
    o = jnp.dot(h, w3b[...], preferred_element_type=jnp.float32)
    out_ref[...] = o + b3_ref[0]


def _mlp(te, xd, W1, b1, W2, b2, W3, b3):
    p, d = xd.shape
    e, _, h = W1.shape
    o = W3.shape[2]
    b1 = b1.reshape(e, 1, h)
    b2 = b2.reshape(e, 1, h)
    b3 = b3.reshape(e, 1, o)
    nt = p // TILE
    grid_spec = pltpu.PrefetchScalarGridSpec(
        num_scalar_prefetch=1,
        grid=(nt,),
        in_specs=[
            pl.BlockSpec((TILE, d), lambda t, te: (t, 0)),
            pl.BlockSpec((1, d, h), lambda t, te: (te[t], 0, 0)),
            pl.BlockSpec((1, 1, h), lambda t, te: (te[t], 0, 0)),
            pl.BlockSpec((1, h, h), lambda t, te: (te[t], 0, 0)),
            pl.BlockSpec((1, 1, h), lambda t, te: (te[t], 0, 0)),
            pl.BlockSpec((1, h, o), lambda t, te: (te[t], 0, 0)),
            pl.BlockSpec((1, 1, o), lambda t, te: (te[t], 0, 0)),
        ],
        out_specs=pl.BlockSpec((TILE, o), lambda t, te: (t, 0)),
        scratch_shapes=[
            pltpu.VMEM((d, h), jnp.bfloat16),
            pltpu.VMEM((h, h), jnp.bfloat16),
            pltpu.VMEM((h, o), jnp.bfloat16),
        ],
    )
    return pl.pallas_call(
        _mlp_body,
        grid_spec=grid_spec,
        out_shape=jax.ShapeDtypeStruct((p, o), jnp.float32),
        compiler_params=pltpu.CompilerParams(
            dimension_semantics=("arbitrary",)),
    )(te, xd, W1, b1, W2, b2, W3, b3)


def _make_combine(n, o, p):
    per_w = n // NW
    ch = 32
    n_ch = per_w // ch
    mesh = plsc.VectorSubcoreMesh(
        core_axis_name="c", subcore_axis_name="s",
        num_cores=NC, num_subcores=NS)

    def body(y_hbm, abi_hbm, abr_hbm, abg_hbm, cnt_hbm, out_hbm,
             abi_v, abr_v, abg_v, cnt_v,
             astart_v, d0i, d1i, buf0, buf1, sem0, sem1):
        wid = lax.axis_index("s") * NC + lax.axis_index("c")
        pltpu.sync_copy(abi_hbm.at[wid], abi_v)
        pltpu.sync_copy(abr_hbm.at[wid], abr_v)
        pltpu.sync_copy(abg_hbm.at[wid], abg_v)
        pltpu.sync_copy(cnt_hbm, cnt_v)
        _slot_dests(cnt_v, abi_v, abr_v, astart_v, d0i, d1i, per_w)
        for c in range(n_ch):
            ca = pltpu.async_copy(y_hbm.at[d0i.at[c]], buf0, sem0)
            cb = pltpu.async_copy(y_hbm.at[d1i.at[c]], buf1, sem1)
            ca.wait()
            cb.wait()

            def row_fma(r, _):
                t = c * ch + r
                g0b = plsc.load_gather(abg_v, [jnp.full((VEC,), t,
                                                        jnp.int32)])
                g1b = plsc.load_gather(abg_v, [jnp.full((VEC,), per_w + t,
                                                        jnp.int32)])
                for j in range(o // VEC):
                    sl = pl.ds(j * VEC, VEC)
                    buf0[r, sl] = buf0[r, sl] * g0b + buf1[r, sl] * g1b
                return 0

            lax.fori_loop(0, ch, row_fma, 0)
            pltpu.sync_copy(
                buf0, out_hbm.at[pl.ds(wid * per_w + c * ch, ch)])

    return pl.kernel(
        body,
        out_type=jax.ShapeDtypeStruct((n, o), jnp.float32),
        mesh=mesh,
        compiler_params=pltpu.CompilerParams(needs_layout_passes=False),
        scratch_types=[
            pltpu.VMEM((2 * per_w,), jnp.int32),
            pltpu.VMEM((2 * per_w,), jnp.int32),
            pltpu.VMEM((2 * per_w,), jnp.float32),
            pltpu.VMEM((VEC,), jnp.int32),
            pltpu.VMEM((VEC,), jnp.int32),
            pltpu.VMEM((n_ch, ch), jnp.int32),
            pltpu.VMEM((n_ch, ch), jnp.int32),
            pltpu.VMEM((ch, o), jnp.float32),
            pltpu.VMEM((ch, o), jnp.float32),
            pltpu.SemaphoreType.DMA,
            pltpu.SemaphoreType.DMA,
        ],
    )


def kernel(x, Wg, bg, W1, b1, W2, b2, W3, b3):
    n, d = x.shape
    e = Wg.shape[1]
    k = 2
    p = n * k + e * TILE
    o = W3.shape[2]

    abi, abr, abg, cnt16, te64 = _gating(x, Wg, bg)
    cnt16 = cnt16.reshape(VEC)
    nt = p // TILE
    te = te64.reshape(64)[:nt]

    xd = _make_dispatch(n, d, p)(x, abi, abr, cnt16)

    y = _mlp(te, xd, W1, b1, W2, b2, W3, b3)

    return _make_combine(n, o, p)(y, abi, abr, abg, cnt16)
